# baseline (speedup 1.0000x reference)
import jax
import jax.numpy as jnp
from jax import lax
from jax.experimental import pallas as pl
from jax.experimental.pallas import tpu as pltpu
from jax.experimental.pallas import tpu_sc as plsc

DIM = 64
NUM_CTX = 5
NC = 2
NS = 16
NW = NC * NS
CB = 128


def _make_sc_body(nchunk):
    def body(tgt_i, ctx_i, tgt_tab, ctx_tab, out,
             tgt_idx_v, ctx_idx_v, tgt_a, tgt_b, ctx_a, ctx_b, out_v,
             sem_a, sem_b):
        w = lax.axis_index("s") * NC + lax.axis_index("c")
        lanes = lax.iota(jnp.int32, 16)
        pltpu.sync_copy(tgt_i.at[w], tgt_idx_v)
        pltpu.sync_copy(ctx_i.at[w], ctx_idx_v)

        tgt_bufs = (tgt_a, tgt_b)
        ctx_bufs = (ctx_a, ctx_b)
        sems = (sem_a, sem_b)

        def fire(k):
            par = k % 2
            waits = [pltpu.async_copy(
                tgt_tab.at[tgt_idx_v.at[k]], tgt_bufs[par], sems[par])]
            for c in range(NUM_CTX):
                waits.append(pltpu.async_copy(
                    ctx_tab.at[ctx_idx_v.at[k, c]],
                    ctx_bufs[par].at[pl.ds(c * CB, CB)], sems[par]))
            return waits

        pending = fire(0)
        for k in range(nchunk):
            for h in pending:
                h.wait()
            if k + 1 < nchunk:
                pending = fire(k + 1)
            tgt_rows = tgt_bufs[k % 2]
            ctx_rows = ctx_bufs[k % 2]
            ks = jnp.full((16,), k, jnp.int32)

            def bstep(b, carry, k=k, tgt_rows=tgt_rows, ctx_rows=ctx_rows,
                      ks=ks):
                wv = [tgt_rows[b, pl.ds(16 * i, 16)] for i in range(4)]
                vec = jnp.zeros((16,), jnp.float32)
                for s in range(NUM_CTX):
                    p = b * NUM_CTX + s
                    acc = wv[0] * ctx_rows[p, pl.ds(0, 16)]
                    for i in range(1, 4):
                        acc = acc + wv[i] * ctx_rows[p, pl.ds(16 * i, 16)]
                    vec = jnp.where(lanes == s, jnp.sum(acc), vec)
                plsc.store_scatter(out_v, [ks, b * NUM_CTX + lanes], vec,
                                   mask=lanes < NUM_CTX)
                return carry

            lax.fori_loop(0, CB, bstep, 0)

        pltpu.sync_copy(out_v, out.at[w])

    return body


def kernel(target, context, target_table, context_table):
    batch, num_ctx = context.shape
    assert num_ctx == NUM_CTX and batch % (NW * CB) == 0
    nchunk = batch // (NW * CB)
    ppc = CB * NUM_CTX

    tgt_i = target.astype(jnp.int32).reshape(NW, nchunk, CB)
    ctx_i = context.astype(jnp.int32).reshape(NW, nchunk, NUM_CTX, CB)

    mesh = plsc.VectorSubcoreMesh(core_axis_name="c", subcore_axis_name="s")
    grid_kernel = pl.kernel(
        _make_sc_body(nchunk),
        out_type=jax.ShapeDtypeStruct((NW, nchunk, ppc), jnp.float32),
        mesh=mesh,
        scratch_types=[
            pltpu.VMEM((nchunk, CB), jnp.int32),
            pltpu.VMEM((nchunk, NUM_CTX, CB), jnp.int32),
            pltpu.VMEM((CB, DIM), jnp.float32),
            pltpu.VMEM((CB, DIM), jnp.float32),
            pltpu.VMEM((NUM_CTX * CB, DIM), jnp.float32),
            pltpu.VMEM((NUM_CTX * CB, DIM), jnp.float32),
            pltpu.VMEM((nchunk, ppc), jnp.float32),
            pltpu.SemaphoreType.DMA,
            pltpu.SemaphoreType.DMA,
        ],
        compiler_params=pltpu.CompilerParams(
            needs_layout_passes=False, use_tc_tiling_on_sc=False),
    )
    out = grid_kernel(tgt_i, ctx_i, target_table, context_table)
    return out.reshape(batch, NUM_CTX)


# speedup vs baseline: 1.2500x; 1.2500x over previous
import jax
import jax.numpy as jnp
from jax import lax
from jax.experimental import pallas as pl
from jax.experimental.pallas import tpu as pltpu
from jax.experimental.pallas import tpu_sc as plsc

DIM = 64
NUM_CTX = 5
NC = 2
NS = 16
NW = NC * NS
CB = 128
VB = 16384
HB = VB // 2


QB = VB // 4
WPR = DIM // 2


def _tc_pack_body(x_ref, o_ref):
    r = lax.broadcasted_iota(jnp.int32, (WPR, WPR), 0)
    c = lax.broadcasted_iota(jnp.int32, (WPR, WPR), 1)
    eye = (r == c).astype(jnp.bfloat16)

    def tpose(xb):
        return lax.dot_general(xb, eye, (((0,), (0,)), ((), ())),
                               preferred_element_type=jnp.float32)

    lo = tpose(x_ref[0:WPR, :].astype(jnp.bfloat16))
    hi = tpose(x_ref[WPR:DIM, :].astype(jnp.bfloat16))
    lu = lax.bitcast_convert_type(lo, jnp.uint32) >> 16
    hu = lax.bitcast_convert_type(hi, jnp.uint32)
    wv = lax.bitcast_convert_type(lu | hu, jnp.float32)
    for q in range(4):
        o_ref[:, WPR * q:WPR * (q + 1)] = wv[QB * q:QB * (q + 1), :]


def _tc_pack(ct, grid):
    return pl.pallas_call(
        _tc_pack_body,
        grid=(grid,),
        in_specs=[pl.BlockSpec((DIM, VB), lambda i: (0, i))],
        out_specs=pl.BlockSpec((QB, 4 * WPR), lambda i: (i, 0)),
        out_shape=jax.ShapeDtypeStruct((grid * QB, 4 * WPR), jnp.float32),
        compiler_params=pltpu.CompilerParams(
            dimension_semantics=("arbitrary",)),
    )(ct)


def _tgt_gather_body(tgt_i, tgt_tab, out, idx_v, rows_v, sem):
    w = lax.axis_index("s") * NC + lax.axis_index("c")
    pltpu.sync_copy(tgt_i.at[w], idx_v)
    nchunk = idx_v.shape[0]
    waits = [pltpu.async_copy(tgt_tab.at[idx_v.at[k]],
                              rows_v.at[k], sem) for k in range(nchunk)]
    for h in waits:
        h.wait()
    pltpu.sync_copy(rows_v, out.at[w])


def _make_sc_body(nchunk):
    def body(ctx_i, tgt_rows_hbm, ctx_tab, out,
             ctx_idx_v, tgt_a, tgt_b, ctx_a, ctx_b, out_v, sem_a, sem_b):
        w = lax.axis_index("s") * NC + lax.axis_index("c")
        lanes = lax.iota(jnp.int32, 16)
        pltpu.sync_copy(ctx_i.at[w], ctx_idx_v)

        tgt_bufs = (tgt_a, tgt_b)
        ctx_bufs = (ctx_a, ctx_b)
        sems = (sem_a, sem_b)

        def fire(k):
            par = k % 2
            waits = [pltpu.async_copy(
                tgt_rows_hbm.at[w, k], tgt_bufs[par], sems[par])]
            for c in range(NUM_CTX):
                waits.append(pltpu.async_copy(
                    ctx_tab.at[ctx_idx_v.at[k, c]],
                    ctx_bufs[par].at[pl.ds(c * CB, CB)], sems[par]))
            return waits

        def unpack2(row_ref, r):
            a0, b0 = plsc.unpack(plsc.bitcast(row_ref[r, pl.ds(0, 16)],
                                              jnp.bfloat16),
                                 format=plsc.PackFormat.INTERLEAVED)
            a1, b1 = plsc.unpack(plsc.bitcast(row_ref[r, pl.ds(16, 16)],
                                              jnp.bfloat16),
                                 format=plsc.PackFormat.INTERLEAVED)
            return (a0, a1, b0, b1)

        pending = fire(0)
        for k in range(nchunk):
            for h in pending:
                h.wait()
            if k + 1 < nchunk:
                pending = fire(k + 1)
            tgt_rows = tgt_bufs[k % 2]
            ctx_rows = ctx_bufs[k % 2]
            ks = jnp.full((16,), k, jnp.int32)

            def bstep(b, carry, k=k, tgt_rows=tgt_rows, ctx_rows=ctx_rows,
                      ks=ks):
                wv = [tgt_rows[b, pl.ds(16 * i, 16)] for i in range(4)]
                vec = jnp.zeros((16,), jnp.float32)
                for s in range(NUM_CTX):
                    p = b * NUM_CTX + s
                    xv = unpack2(ctx_rows, p)
                    acc = wv[0] * xv[0]
                    for i in range(1, 4):
                        acc = acc + wv[i] * xv[i]
                    vec = jnp.where(lanes == s, jnp.sum(acc), vec)
                plsc.store_scatter(out_v, [ks, b * NUM_CTX + lanes], vec,
                                   mask=lanes < NUM_CTX)
                return carry

            lax.fori_loop(0, CB, bstep, 0)

        pltpu.sync_copy(out_v, out.at[w])

    return body


def kernel(target, context, target_table, context_table):
    batch, num_ctx = context.shape
    vocab = target_table.shape[0]
    assert num_ctx == NUM_CTX and batch % (NW * CB) == 0
    nchunk = batch // (NW * CB)
    ppc = CB * NUM_CTX
    grid = (vocab + VB - 1) // VB

    tgt_i = target.astype(jnp.int32).reshape(NW, nchunk, CB)
    mesh = plsc.VectorSubcoreMesh(core_axis_name="c", subcore_axis_name="s")
    tgt_rows = pl.kernel(
        _tgt_gather_body,
        out_type=jax.ShapeDtypeStruct((NW, nchunk, CB, DIM), jnp.float32),
        mesh=mesh,
        scratch_types=[
            pltpu.VMEM((nchunk, CB), jnp.int32),
            pltpu.VMEM((nchunk, CB, DIM), jnp.float32),
            pltpu.SemaphoreType.DMA,
        ],
        compiler_params=pltpu.CompilerParams(
            needs_layout_passes=False, use_tc_tiling_on_sc=False),
    )(tgt_i, target_table)

    cpk = _tc_pack(context_table.T, grid).reshape(4 * grid * QB, WPR)

    sh_vb = VB.bit_length() - 1
    sh_qb = QB.bit_length() - 1

    def addr(v):
        v = v.astype(jnp.int32)
        return (v >> sh_vb) * VB + (v & (QB - 1)) * 4 + ((v >> sh_qb) & 3)

    ctx_i = addr(context).reshape(NW, nchunk, NUM_CTX, CB)

    grid_kernel = pl.kernel(
        _make_sc_body(nchunk),
        out_type=jax.ShapeDtypeStruct((NW, nchunk, ppc), jnp.float32),
        mesh=mesh,
        scratch_types=[
            pltpu.VMEM((nchunk, NUM_CTX, CB), jnp.int32),
            pltpu.VMEM((CB, DIM), jnp.float32),
            pltpu.VMEM((CB, DIM), jnp.float32),
            pltpu.VMEM((NUM_CTX * CB, WPR), jnp.float32),
            pltpu.VMEM((NUM_CTX * CB, WPR), jnp.float32),
            pltpu.VMEM((nchunk, ppc), jnp.float32),
            pltpu.SemaphoreType.DMA,
            pltpu.SemaphoreType.DMA,
        ],
        compiler_params=pltpu.CompilerParams(
            needs_layout_passes=False, use_tc_tiling_on_sc=False),
    )
    out = grid_kernel(ctx_i, tgt_rows, cpk)
    return out.reshape(batch, NUM_CTX)


# speedup vs baseline: 2.0609x; 1.6486x over previous
import jax
import jax.numpy as jnp
from jax import lax
from jax.experimental import pallas as pl
from jax.experimental.pallas import tpu as pltpu
from jax.experimental.pallas import tpu_sc as plsc

DIM = 64
NUM_CTX = 5
NC = 2
NS = 16
NW = NC * NS
CB = 128
VB = 8192
QB = VB // 4
WPR = DIM // 2


def _tc_pack_body(xt_ref, xc_ref, ot_ref, oc_ref):
    r = lax.broadcasted_iota(jnp.int32, (WPR, WPR), 0)
    c = lax.broadcasted_iota(jnp.int32, (WPR, WPR), 1)
    eye = (r == c).astype(jnp.bfloat16)

    def tpose(xb):
        return lax.dot_general(xb, eye, (((0,), (0,)), ((), ())),
                               preferred_element_type=jnp.float32)

    for x_ref, o_ref in ((xt_ref, ot_ref), (xc_ref, oc_ref)):
        lo = tpose(x_ref[0:WPR, :].astype(jnp.bfloat16))
        hi = tpose(x_ref[WPR:DIM, :].astype(jnp.bfloat16))
        lu = lax.bitcast_convert_type(lo, jnp.uint32) >> 16
        hu = lax.bitcast_convert_type(hi, jnp.uint32)
        wv = lax.bitcast_convert_type(lu | hu, jnp.float32)
        for q in range(4):
            o_ref[:, WPR * q:WPR * (q + 1)] = wv[QB * q:QB * (q + 1), :]


def _tc_pack(tt, ct, grid):
    spec_in = pl.BlockSpec((DIM, VB), lambda i: (0, i))
    spec_out = pl.BlockSpec((QB, 4 * WPR), lambda i: (i, 0))
    out_sds = jax.ShapeDtypeStruct((grid * QB, 4 * WPR), jnp.float32)
    return pl.pallas_call(
        _tc_pack_body,
        grid=(grid,),
        in_specs=[spec_in, spec_in],
        out_specs=[spec_out, spec_out],
        out_shape=[out_sds, out_sds],
        compiler_params=pltpu.CompilerParams(
            dimension_semantics=("arbitrary",)),
    )(tt, ct)


def _make_sc_body(nchunk):
    def body(tgt_i, ctx_i, tgt_tab, ctx_tab, out,
             tgt_idx_v, ctx_idx_v, tgt_a, tgt_b, ctx_a, ctx_b, out_v,
             sem_a, sem_b):
        w = lax.axis_index("s") * NC + lax.axis_index("c")
        lanes = lax.iota(jnp.int32, 16)
        pltpu.sync_copy(tgt_i.at[w], tgt_idx_v)
        pltpu.sync_copy(ctx_i.at[w], ctx_idx_v)

        tgt_bufs = (tgt_a, tgt_b)
        ctx_bufs = (ctx_a, ctx_b)
        sems = (sem_a, sem_b)

        def fire(k):
            par = k % 2
            waits = [pltpu.async_copy(
                tgt_tab.at[tgt_idx_v.at[k]], tgt_bufs[par], sems[par])]
            for c in range(NUM_CTX):
                waits.append(pltpu.async_copy(
                    ctx_tab.at[ctx_idx_v.at[k, c]],
                    ctx_bufs[par].at[pl.ds(c * CB, CB)], sems[par]))
            return waits

        def unpack2(row_ref, r):
            a0, b0 = plsc.unpack(plsc.bitcast(row_ref[r, pl.ds(0, 16)],
                                              jnp.bfloat16),
                                 format=plsc.PackFormat.INTERLEAVED)
            a1, b1 = plsc.unpack(plsc.bitcast(row_ref[r, pl.ds(16, 16)],
                                              jnp.bfloat16),
                                 format=plsc.PackFormat.INTERLEAVED)
            return (a0, a1, b0, b1)

        pending = fire(0)
        for k in range(nchunk):
            for h in pending:
                h.wait()
            if k + 1 < nchunk:
                pending = fire(k + 1)
            tgt_rows = tgt_bufs[k % 2]
            ctx_rows = ctx_bufs[k % 2]
            ks = jnp.full((16,), k, jnp.int32)

            def bstep(b, carry, k=k, tgt_rows=tgt_rows, ctx_rows=ctx_rows,
                      ks=ks):
                wv = unpack2(tgt_rows, b)
                vec = jnp.zeros((16,), jnp.float32)
                for s in range(NUM_CTX):
                    p = b * NUM_CTX + s
                    xv = unpack2(ctx_rows, p)
                    acc = wv[0] * xv[0]
                    for i in range(1, 4):
                        acc = acc + wv[i] * xv[i]
                    vec = jnp.where(lanes == s, jnp.sum(acc), vec)
                plsc.store_scatter(out_v, [ks, b * NUM_CTX + lanes], vec,
                                   mask=lanes < NUM_CTX)
                return carry

            lax.fori_loop(0, CB, bstep, 0)

        pltpu.sync_copy(out_v, out.at[w])

    return body


def kernel(target, context, target_table, context_table):
    batch, num_ctx = context.shape
    vocab = target_table.shape[0]
    assert num_ctx == NUM_CTX and batch % (NW * CB) == 0
    nchunk = batch // (NW * CB)
    ppc = CB * NUM_CTX
    grid = (vocab + VB - 1) // VB

    tpk, cpk = _tc_pack(target_table.T, context_table.T, grid)
    nrows = 4 * grid * QB
    tpk = tpk.reshape(nrows, WPR)
    cpk = cpk.reshape(nrows, WPR)

    sh_vb = VB.bit_length() - 1
    sh_qb = QB.bit_length() - 1

    def addr(v):
        v = v.astype(jnp.int32)
        return (v >> sh_vb) * VB + (v & (QB - 1)) * 4 + ((v >> sh_qb) & 3)

    tgt_i = addr(target).reshape(NW, nchunk, CB)
    ctx_i = addr(context).reshape(NW, nchunk, NUM_CTX, CB)

    mesh = plsc.VectorSubcoreMesh(core_axis_name="c", subcore_axis_name="s")
    grid_kernel = pl.kernel(
        _make_sc_body(nchunk),
        out_type=jax.ShapeDtypeStruct((NW, nchunk, ppc), jnp.float32),
        mesh=mesh,
        scratch_types=[
            pltpu.VMEM((nchunk, CB), jnp.int32),
            pltpu.VMEM((nchunk, NUM_CTX, CB), jnp.int32),
            pltpu.VMEM((CB, WPR), jnp.float32),
            pltpu.VMEM((CB, WPR), jnp.float32),
            pltpu.VMEM((NUM_CTX * CB, WPR), jnp.float32),
            pltpu.VMEM((NUM_CTX * CB, WPR), jnp.float32),
            pltpu.VMEM((nchunk, ppc), jnp.float32),
            pltpu.SemaphoreType.DMA,
            pltpu.SemaphoreType.DMA,
        ],
        compiler_params=pltpu.CompilerParams(
            needs_layout_passes=False, use_tc_tiling_on_sc=False),
    )
    out = grid_kernel(tgt_i, ctx_i, tpk, cpk)
    return out.reshape(batch, NUM_CTX)


# speedup vs baseline: 2.0693x; 1.0041x over previous
import jax
import jax.numpy as jnp
from jax import lax
from jax.experimental import pallas as pl
from jax.experimental.pallas import tpu as pltpu
from jax.experimental.pallas import tpu_sc as plsc

DIM = 64
NUM_CTX = 5
NC = 2
NS = 16
NW = NC * NS
CB = 128
VB = 16384
QB = VB // 4
WPR = DIM // 2


def _tc_pack_body(xt_ref, xc_ref, ot_ref, oc_ref):
    r = lax.broadcasted_iota(jnp.int32, (WPR, WPR), 0)
    c = lax.broadcasted_iota(jnp.int32, (WPR, WPR), 1)
    eye = (r == c).astype(jnp.bfloat16)

    def tpose(xb):
        return lax.dot_general(xb, eye, (((0,), (0,)), ((), ())),
                               preferred_element_type=jnp.float32)

    for x_ref, o_ref in ((xt_ref, ot_ref), (xc_ref, oc_ref)):
        lo = tpose(x_ref[0:WPR, :].astype(jnp.bfloat16))
        hi = tpose(x_ref[WPR:DIM, :].astype(jnp.bfloat16))
        lu = lax.bitcast_convert_type(lo, jnp.uint32) >> 16
        hu = lax.bitcast_convert_type(hi, jnp.uint32)
        wv = lax.bitcast_convert_type(lu | hu, jnp.float32)
        for q in range(4):
            o_ref[:, WPR * q:WPR * (q + 1)] = wv[QB * q:QB * (q + 1), :]


def _tc_pack(tt, ct, grid):
    spec_in = pl.BlockSpec((DIM, VB), lambda i: (0, i))
    spec_out = pl.BlockSpec((QB, 4 * WPR), lambda i: (i, 0))
    out_sds = jax.ShapeDtypeStruct((grid * QB, 4 * WPR), jnp.float32)
    return pl.pallas_call(
        _tc_pack_body,
        grid=(grid,),
        in_specs=[spec_in, spec_in],
        out_specs=[spec_out, spec_out],
        out_shape=[out_sds, out_sds],
        compiler_params=pltpu.CompilerParams(
            dimension_semantics=("arbitrary",)),
    )(tt, ct)


def _make_sc_body(nchunk):
    def body(tgt_i, ctx_i, tgt_tab, ctx_tab, out,
             tgt_idx_v, ctx_idx_v, tgt_a, tgt_b, ctx_a, ctx_b, out_v,
             sem_a, sem_b):
        w = lax.axis_index("s") * NC + lax.axis_index("c")
        lanes = lax.iota(jnp.int32, 16)
        pltpu.sync_copy(tgt_i.at[w], tgt_idx_v)
        pltpu.sync_copy(ctx_i.at[w], ctx_idx_v)

        tgt_bufs = (tgt_a, tgt_b)
        ctx_bufs = (ctx_a, ctx_b)
        sems = (sem_a, sem_b)

        def fire(k):
            par = k % 2
            waits = [pltpu.async_copy(
                tgt_tab.at[tgt_idx_v.at[k]], tgt_bufs[par], sems[par])]
            for c in range(NUM_CTX):
                waits.append(pltpu.async_copy(
                    ctx_tab.at[ctx_idx_v.at[k, c]],
                    ctx_bufs[par].at[pl.ds(c * CB, CB)], sems[par]))
            return waits

        def unpack2(row_ref, r):
            a0, b0 = plsc.unpack(plsc.bitcast(row_ref[r, pl.ds(0, 16)],
                                              jnp.bfloat16),
                                 format=plsc.PackFormat.INTERLEAVED)
            a1, b1 = plsc.unpack(plsc.bitcast(row_ref[r, pl.ds(16, 16)],
                                              jnp.bfloat16),
                                 format=plsc.PackFormat.INTERLEAVED)
            return (a0, a1, b0, b1)

        pending = fire(0)
        for k in range(nchunk):
            for h in pending:
                h.wait()
            if k + 1 < nchunk:
                pending = fire(k + 1)
            tgt_rows = tgt_bufs[k % 2]
            ctx_rows = ctx_bufs[k % 2]
            ks = jnp.full((16,), k, jnp.int32)

            def bstep(b, carry, k=k, tgt_rows=tgt_rows, ctx_rows=ctx_rows,
                      ks=ks):
                wv = unpack2(tgt_rows, b)
                vec = jnp.zeros((16,), jnp.float32)
                for s in range(NUM_CTX):
                    p = b * NUM_CTX + s
                    xv = unpack2(ctx_rows, p)
                    acc = wv[0] * xv[0]
                    for i in range(1, 4):
                        acc = acc + wv[i] * xv[i]
                    vec = jnp.where(lanes == s, jnp.sum(acc), vec)
                plsc.store_scatter(out_v, [ks, b * NUM_CTX + lanes], vec,
                                   mask=lanes < NUM_CTX)
                return carry

            lax.fori_loop(0, CB, bstep, 0)

        pltpu.sync_copy(out_v, out.at[w])

    return body


def kernel(target, context, target_table, context_table):
    batch, num_ctx = context.shape
    vocab = target_table.shape[0]
    assert num_ctx == NUM_CTX and batch % (NW * CB) == 0
    nchunk = batch // (NW * CB)
    ppc = CB * NUM_CTX
    grid = (vocab + VB - 1) // VB

    tpk, cpk = _tc_pack(target_table.T, context_table.T, grid)
    nrows = 4 * grid * QB
    tpk = tpk.reshape(nrows, WPR)
    cpk = cpk.reshape(nrows, WPR)

    sh_vb = VB.bit_length() - 1
    sh_qb = QB.bit_length() - 1

    def addr(v):
        v = v.astype(jnp.int32)
        return (v >> sh_vb) * VB + (v & (QB - 1)) * 4 + ((v >> sh_qb) & 3)

    tgt_i = addr(target).reshape(NW, nchunk, CB)
    ctx_i = addr(context).reshape(NW, nchunk, NUM_CTX, CB)

    mesh = plsc.VectorSubcoreMesh(core_axis_name="c", subcore_axis_name="s")
    grid_kernel = pl.kernel(
        _make_sc_body(nchunk),
        out_type=jax.ShapeDtypeStruct((NW, nchunk, ppc), jnp.float32),
        mesh=mesh,
        scratch_types=[
            pltpu.VMEM((nchunk, CB), jnp.int32),
            pltpu.VMEM((nchunk, NUM_CTX, CB), jnp.int32),
            pltpu.VMEM((CB, WPR), jnp.float32),
            pltpu.VMEM((CB, WPR), jnp.float32),
            pltpu.VMEM((NUM_CTX * CB, WPR), jnp.float32),
            pltpu.VMEM((NUM_CTX * CB, WPR), jnp.float32),
            pltpu.VMEM((nchunk, ppc), jnp.float32),
            pltpu.SemaphoreType.DMA,
            pltpu.SemaphoreType.DMA,
        ],
        compiler_params=pltpu.CompilerParams(
            needs_layout_passes=False, use_tc_tiling_on_sc=False),
    )
    out = grid_kernel(tgt_i, ctx_i, tpk, cpk)
    return out.reshape(batch, NUM_CTX)
